# baseline (speedup 1.0000x reference)
import functools

import jax
import jax.numpy as jnp
from jax import lax
from jax.experimental import pallas as pl
from jax.experimental.pallas import tpu as pltpu
from jax.experimental.pallas import tpu_sc as plsc

_NUM_CORES = 2
_NUM_SUBCORES = 16
_NUM_WORKERS = _NUM_CORES * _NUM_SUBCORES
_L = 16

_B, _T, _U, _X, _Y = 2, 12, 8, 112, 112
_Y2 = 2 * _Y
_PLANES = _B * _T * _U
_QUARTERS = 4
_NT = _PLANES * _QUARTERS
_ROWS_IN = _X // _QUARTERS
_ROWS_OUT = 2 * _ROWS_IN
_GPR = _Y // _L
_TASK_VALS = _ROWS_IN * _Y
_TASKS_PER_WORKER = _NT // _NUM_WORKERS


def _unpool_body(vr_hbm, vi_hbm, idx_hbm, outr_hbm, outi_hbm,
                 idx0, idx1, vr0, vr1, vi0, vi1,
                 outr0, outr1, outi0, outi1,
                 sem_i0, sem_i1, sem_v0, sem_v1, sem_w0, sem_w1,
                 sem_or0, sem_or1, sem_oi0, sem_oi1):
    wid = lax.axis_index("s") * _NUM_CORES + lax.axis_index("c")
    outr_flat = outr_hbm.reshape(_NT, _ROWS_OUT, _Y2)
    outi_flat = outi_hbm.reshape(_NT, _ROWS_OUT, _Y2)

    idx_b = (idx0, idx1)
    vr_b = (vr0, vr1)
    vi_b = (vi0, vi1)
    outr_b = (outr0, outr1)
    outi_b = (outi0, outi1)
    sem_i = (sem_i0, sem_i1)
    sem_v = (sem_v0, sem_v1)
    sem_w = (sem_w0, sem_w1)
    sem_or = (sem_or0, sem_or1)
    sem_oi = (sem_oi0, sem_oi1)

    zeros = jnp.zeros((_L,), jnp.float32)

    def _start_in(k):
        t = wid * _TASKS_PER_WORKER + k
        s = k % 2
        return (
            pltpu.async_copy(idx_hbm.at[t], idx_b[s], sem_i[s]),
            pltpu.async_copy(vr_hbm.at[t], vr_b[s], sem_v[s]),
            pltpu.async_copy(vi_hbm.at[t], vi_b[s], sem_w[s]),
        )

    pending_in = _start_in(0)
    pending_out = [None, None]

    for k in range(_TASKS_PER_WORKER):
        t = wid * _TASKS_PER_WORKER + k
        s = k % 2
        q = k % 4

        for c in pending_in:
            c.wait()
        if k + 1 < _TASKS_PER_WORKER:
            pending_in = _start_in(k + 1)

        if pending_out[s] is not None:
            for c in pending_out[s]:
                c.wait()

        idx_v, vr_v, vi_v = idx_b[s], vr_b[s], vi_b[s]
        outr_v, outi_v = outr_b[s], outi_b[s]

        def _zero(r, carry):
            for u in range(_GPR * 2):
                outr_v[r, pl.ds(u * _L, _L)] = zeros
                outi_v[r, pl.ds(u * _L, _L)] = zeros
            return carry

        lax.fori_loop(0, _ROWS_OUT, _zero, 0, unroll=False)

        def _scat(il, carry):
            rowbase = (q * _ROWS_IN + il) * (2 * _Y2)
            for u in range(_GPR):
                o = il * _Y + u * _L
                rem = idx_v[pl.ds(o, _L)] - rowbase
                di = jnp.where(rem >= _Y2, 1, 0)
                c = rem - di * _Y2
                r = 2 * il + di
                plsc.store_scatter(outr_v, [r, c], vr_v[pl.ds(o, _L)])
                plsc.store_scatter(outi_v, [r, c], vi_v[pl.ds(o, _L)])
            return carry

        lax.fori_loop(0, _ROWS_IN, _scat, 0, unroll=False)

        pending_out[s] = (
            pltpu.async_copy(outr_v, outr_flat.at[t], sem_or[s]),
            pltpu.async_copy(outi_v, outi_flat.at[t], sem_oi[s]),
        )

    for s in range(2):
        if pending_out[s] is not None:
            for c in pending_out[s]:
                c.wait()


_OUT5D = (_B, _T, _U, 2 * _X, 2 * _Y)

_unpool_sc = functools.partial(
    pl.kernel,
    out_type=(
        jax.ShapeDtypeStruct(_OUT5D, jnp.float32),
        jax.ShapeDtypeStruct(_OUT5D, jnp.float32),
    ),
    mesh=plsc.VectorSubcoreMesh(core_axis_name="c", subcore_axis_name="s"),
    compiler_params=pltpu.CompilerParams(needs_layout_passes=False),
    scratch_types=[
        pltpu.VMEM((_TASK_VALS,), jnp.int32),
        pltpu.VMEM((_TASK_VALS,), jnp.int32),
        pltpu.VMEM((_TASK_VALS,), jnp.float32),
        pltpu.VMEM((_TASK_VALS,), jnp.float32),
        pltpu.VMEM((_TASK_VALS,), jnp.float32),
        pltpu.VMEM((_TASK_VALS,), jnp.float32),
        pltpu.VMEM((_ROWS_OUT, _Y2), jnp.float32),
        pltpu.VMEM((_ROWS_OUT, _Y2), jnp.float32),
        pltpu.VMEM((_ROWS_OUT, _Y2), jnp.float32),
        pltpu.VMEM((_ROWS_OUT, _Y2), jnp.float32),
        pltpu.SemaphoreType.DMA,
        pltpu.SemaphoreType.DMA,
        pltpu.SemaphoreType.DMA,
        pltpu.SemaphoreType.DMA,
        pltpu.SemaphoreType.DMA,
        pltpu.SemaphoreType.DMA,
        pltpu.SemaphoreType.DMA,
        pltpu.SemaphoreType.DMA,
        pltpu.SemaphoreType.DMA,
        pltpu.SemaphoreType.DMA,
    ],
)(_unpool_body)


def kernel(input_real, input_imag, pooling_indices):
    vr = input_real.reshape(_NT, _TASK_VALS)
    vi = input_imag.reshape(_NT, _TASK_VALS)
    idx = pooling_indices.reshape(_NT, _TASK_VALS)
    outr, outi = _unpool_sc(vr, vi, idx)
    return lax.complex(outr, outi)
